# baseline (speedup 1.0000x reference)
import jax
import jax.numpy as jnp
from jax import lax
from jax.experimental import pallas as pl
from jax.experimental.pallas import tpu as pltpu

_PRE = 2048
_POST = 500
_THRESH = 0.7
_SUB = 8
_LANE = _PRE // _SUB
_NCOL = 24
_OUTS = 512


def _pair_sup(cx1, cx2, cy1, cy2, ca, rx1, rx2, ry1, ry2, ra):
    ix = jnp.maximum(jnp.minimum(cx2, rx2) - jnp.maximum(cx1, rx1), 0.0)
    iy = jnp.maximum(jnp.minimum(cy2, ry2) - jnp.maximum(cy1, ry1), 0.0)
    inter = ix * iy
    iou = inter / jnp.maximum(ca + ra - inter, 1e-6)
    return (iou > _THRESH).astype(jnp.float32)


def _nms_body(rows_ref, feats_ref, out_ref, keep_ref):
    x1v = feats_ref[0, 0]
    x2v = feats_ref[0, 1]
    y1v = feats_ref[0, 2]
    y2v = feats_ref[0, 3]
    av = feats_ref[0, 4]
    keep_ref[...] = jnp.ones((_SUB, _LANE), jnp.float32)

    for c in range(_SUB):
        base = c * _LANE
        fc = rows_ref[0, base:base + _LANE, :]
        cx1, cx2 = fc[:, 8:9], fc[:, 9:10]
        cy1, cy2 = fc[:, 10:11], fc[:, 11:12]
        ca = fc[:, 12:13]

        d = _pair_sup(cx1, cx2, cy1, cy2, ca,
                      x1v[c:c + 1, :], x2v[c:c + 1, :],
                      y1v[c:c + 1, :], y2v[c:c + 1, :], av[c:c + 1, :])
        tri = (lax.broadcasted_iota(jnp.int32, (_LANE, _LANE), 0)
               < lax.broadcasted_iota(jnp.int32, (_LANE, _LANE), 1))
        d = d * tri.astype(jnp.float32)

        keep0 = keep_ref[c:c + 1, :]

        def fix_cond(state):
            return state[1]

        def fix_body(state):
            k, _ = state
            a = lax.dot(k, d)
            k_new = keep0 * (a < 0.5).astype(jnp.float32)
            return k_new, jnp.any(k_new != k)

        k, _ = lax.while_loop(fix_cond, fix_body,
                              (keep0, jnp.bool_(True)))
        keep_ref[c:c + 1, :] = k

        for r in range(c, _SUB):
            for h in range(2):
                l0 = h * 128
                db = _pair_sup(cx1, cx2, cy1, cy2, ca,
                               x1v[r:r + 1, l0:l0 + 128],
                               x2v[r:r + 1, l0:l0 + 128],
                               y1v[r:r + 1, l0:l0 + 128],
                               y2v[r:r + 1, l0:l0 + 128],
                               av[r:r + 1, l0:l0 + 128])
                if r == c:
                    m = (lax.broadcasted_iota(jnp.int32, (_LANE, 128), 0)
                         < lax.broadcasted_iota(jnp.int32, (_LANE, 128), 1)
                         + l0)
                    db = db * m.astype(jnp.float32)
                sup = lax.dot(k, db)
                keep_ref[r:r + 1, l0:l0 + 128] = (
                    keep_ref[r:r + 1, l0:l0 + 128]
                    * (sup < 0.5).astype(jnp.float32))

    keep = keep_ref[...]
    iu = (lax.broadcasted_iota(jnp.int32, (_LANE, _LANE), 0)
          <= lax.broadcasted_iota(jnp.int32, (_LANE, _LANE), 1))
    cs = lax.dot(keep, iu.astype(jnp.float32))
    tot = cs[:, _LANE - 1:_LANE]
    lo = (lax.broadcasted_iota(jnp.int32, (_SUB, _SUB), 0)
          > lax.broadcasted_iota(jnp.int32, (_SUB, _SUB), 1))
    off = lax.dot(lo.astype(jnp.float32), tot)
    sel = cs + off - 1.0

    i512 = lax.broadcasted_iota(jnp.int32, (_OUTS, 128), 0)
    sel_i = sel.astype(jnp.int32)
    acc = jnp.zeros((_OUTS, _NCOL), jnp.float32)
    for c in range(_PRE // 128):
        r_, l0 = c // 2, (c % 2) * 128
        sel_s = sel_i[r_:r_ + 1, l0:l0 + 128]
        keep_s = keep[r_:r_ + 1, l0:l0 + 128]
        pt = (i512 == sel_s).astype(jnp.float32) * keep_s
        acc = acc + lax.dot(pt, rows_ref[0, c * 128:(c + 1) * 128, :],
                            precision=lax.Precision.HIGHEST)
    out_ref[0] = acc


def kernel(rpn_box_preds, rpn_cls_preds):
    B = rpn_box_preds.shape[0]
    scores_all = jnp.max(rpn_cls_preds, axis=-1)
    labels_all = jnp.argmax(rpn_cls_preds, axis=-1)
    top_scores, top_idx = lax.top_k(scores_all, _PRE)
    tb = jnp.take_along_axis(rpn_box_preds, top_idx[..., None], axis=1)
    tl = jnp.take_along_axis(labels_all, top_idx, axis=1)

    x, y = tb[..., 0], tb[..., 1]
    dx, dy = tb[..., 3], tb[..., 4]
    x1 = x - dx * 0.5
    x2 = x + dx * 0.5
    y1 = y - dy * 0.5
    y2 = y + dy * 0.5
    area = dx * dy
    zc = jnp.zeros_like(x)
    rows = jnp.stack(
        [tb[..., 0], tb[..., 1], tb[..., 2], tb[..., 3], tb[..., 4],
         tb[..., 5], tb[..., 6], zc,
         x1, x2, y1, y2, area, top_scores,
         (tl + 1).astype(jnp.float32), zc,
         zc, zc, zc, zc, zc, zc, zc, zc], axis=-1)
    feats = jnp.stack(
        [x1.reshape(B, _SUB, _LANE), x2.reshape(B, _SUB, _LANE),
         y1.reshape(B, _SUB, _LANE), y2.reshape(B, _SUB, _LANE),
         area.reshape(B, _SUB, _LANE)], axis=1)

    out = pl.pallas_call(
        _nms_body,
        grid=(B,),
        in_specs=[
            pl.BlockSpec((1, _PRE, _NCOL), lambda b: (b, 0, 0)),
            pl.BlockSpec((1, 5, _SUB, _LANE), lambda b: (b, 0, 0, 0)),
        ],
        out_specs=pl.BlockSpec((1, _OUTS, _NCOL), lambda b: (b, 0, 0)),
        out_shape=jax.ShapeDtypeStruct((B, _OUTS, _NCOL), jnp.float32),
        scratch_shapes=[pltpu.VMEM((_SUB, _LANE), jnp.float32)],
        compiler_params=pltpu.CompilerParams(
            dimension_semantics=("parallel",)),
    )(rows, feats)

    rois = out[:, :_POST, 0:7]
    roi_scores = out[:, :_POST, 13]
    roi_labels = out[:, :_POST, 14].astype(jnp.int32)
    return rois, roi_scores, roi_labels
